# Initial kernel scaffold; baseline (speedup 1.0000x reference)
import jax
import jax.numpy as jnp
from jax.experimental import pallas as pl


def kernel(x, feature, mask, W1, W2, W3, W4, b4):
    raise NotImplementedError("write your pallas kernel here")


# speedup vs baseline: 9.7484x; 9.7484x over previous
import functools

import jax
import jax.numpy as jnp
import numpy as np
from jax import lax
from jax.experimental import pallas as pl
from jax.experimental.pallas import tpu as pltpu
from jax.experimental.pallas import tpu_sc as plsc

_B = 8
_KN = 12800
_NB = 230
_NCOV = 26
_NPTS = 256
_HW = 64
_P = _HW * _HW


def _corner_data(pts):
    g = 2.0 * pts - 1.0
    x = ((g[..., 0] + 1.0) * _HW - 1.0) / 2.0
    y = ((g[..., 1] + 1.0) * _HW - 1.0) / 2.0
    x0 = jnp.floor(x)
    x1 = x0 + 1.0
    y0 = jnp.floor(y)
    y1 = y0 + 1.0
    wx1 = x - x0
    wx0 = 1.0 - wx1
    wy1 = y - y0
    wy0 = 1.0 - wy1
    idxs, wts = [], []
    for xi, yi, wx, wy in ((x0, y0, wx0, wy0), (x1, y0, wx1, wy0),
                           (x0, y1, wx0, wy1), (x1, y1, wx1, wy1)):
        valid = ((xi >= 0) & (xi <= _HW - 1) & (yi >= 0) & (yi <= _HW - 1))
        xi_c = jnp.clip(xi, 0, _HW - 1).astype(jnp.int32)
        yi_c = jnp.clip(yi, 0, _HW - 1).astype(jnp.int32)
        idxs.append(yi_c * _HW + xi_c)
        wts.append(jnp.where(valid, wx * wy, 0.0))
    return idxs, wts


def _build_consts():
    key = jax.random.key(42)
    k1, k2 = jax.random.split(key)
    over = jax.random.uniform(k1, (_B, _KN, 2), dtype=jnp.float32)
    coverage = jax.random.uniform(k2, (_B, _NCOV, 2), dtype=jnp.float32)
    ci, cw = _corner_data(over)
    vi, vw = _corner_data(coverage)
    covtab = np.zeros((_B, 32, 128), np.float32)
    cols = [coverage[..., 0], coverage[..., 1]] + \
           [v.astype(jnp.float32) for v in vi] + vw
    for c, v in enumerate(cols):
        covtab[:, 6:32, c] = np.asarray(v)
    cand_f32 = [np.asarray(over[..., 0]).reshape(_B, 100, 128),
                np.asarray(over[..., 1]).reshape(_B, 100, 128)] + \
               [np.asarray(v.astype(jnp.float32)).reshape(_B, 100, 128) for v in ci] + \
               [np.asarray(v).reshape(_B, 100, 128) for v in cw]
    cand_i32 = [np.asarray(v).astype(np.int32) for v in ci]
    cand_w = [np.asarray(v) for v in cw]
    return cand_i32, cand_w, cand_f32, covtab


_CAND_I32, _CAND_W, _CAND_F32, _COVTAB = _build_consts()

_CHUNK = _KN // 4


def _sc_uncertainty(s0, s1, i00, i10, i01, i11, w00, w10, w01, w11):
    mesh = plsc.VectorSubcoreMesh(core_axis_name="c", subcore_axis_name="s")

    @functools.partial(
        pl.kernel, mesh=mesh,
        compiler_params=pltpu.CompilerParams(needs_layout_passes=False),
        out_type=jax.ShapeDtypeStruct((_B, _KN), jnp.float32),
        scratch_types=[
            pltpu.VMEM((_P,), jnp.float32), pltpu.VMEM((_P,), jnp.float32),
            pltpu.VMEM((_CHUNK,), jnp.int32), pltpu.VMEM((_CHUNK,), jnp.int32),
            pltpu.VMEM((_CHUNK,), jnp.int32), pltpu.VMEM((_CHUNK,), jnp.int32),
            pltpu.VMEM((_CHUNK,), jnp.float32), pltpu.VMEM((_CHUNK,), jnp.float32),
            pltpu.VMEM((_CHUNK,), jnp.float32), pltpu.VMEM((_CHUNK,), jnp.float32),
            pltpu.VMEM((_CHUNK,), jnp.float32),
        ])
    def k(s0_h, s1_h, i00_h, i10_h, i01_h, i11_h, w00_h, w10_h, w01_h, w11_h,
          u_h, s0_v, s1_v, i00_v, i10_v, i01_v, i11_v,
          w00_v, w10_v, w01_v, w11_v, u_v):
        wid = lax.axis_index("s") * 2 + lax.axis_index("c")
        b = wid // 4
        off = (wid % 4) * _CHUNK
        pltpu.sync_copy(s0_h.at[b], s0_v)
        pltpu.sync_copy(s1_h.at[b], s1_v)
        for src, dst in ((i00_h, i00_v), (i10_h, i10_v), (i01_h, i01_v),
                         (i11_h, i11_v), (w00_h, w00_v), (w10_h, w10_v),
                         (w01_h, w01_v), (w11_h, w11_v)):
            pltpu.sync_copy(src.at[b, pl.ds(off, _CHUNK)], dst)

        def body(t, carry):
            s = pl.ds(t * 16, 16)
            j00, j10, j01, j11 = i00_v[s], i10_v[s], i01_v[s], i11_v[s]
            a = plsc.load_gather(s0_v, [j00])
            bq = plsc.load_gather(s0_v, [j10])
            c = plsc.load_gather(s0_v, [j01])
            d = plsc.load_gather(s0_v, [j11])
            e = plsc.load_gather(s1_v, [j00])
            f = plsc.load_gather(s1_v, [j10])
            g = plsc.load_gather(s1_v, [j01])
            h = plsc.load_gather(s1_v, [j11])
            q00, q10, q01, q11 = w00_v[s], w10_v[s], w01_v[s], w11_v[s]
            og0 = ((a * q00 + bq * q10) + c * q01) + d * q11
            og1 = ((e * q00 + f * q10) + g * q01) + h * q11
            u_v[s] = og1 - og0
            return carry

        lax.fori_loop(0, _CHUNK // 16, body, 0)
        pltpu.sync_copy(u_v, u_h.at[b, pl.ds(off, _CHUNK)])

    return k(s0, s1, i00, i10, i01, i11, w00, w10, w01, w11)


def _tc_body(u_ref, cx_ref, cy_ref, f00_ref, f10_ref, f01_ref, f11_ref,
             g00_ref, g10_ref, g01_ref, g11_ref, cov_ref, mft_ref,
             w1_ref, w2_ref, w3_ref, w4_ref, b4_ref,
             pts_ref, rend_ref, uwork, tab):
    uwork[...] = u_ref[0]
    lin = (lax.broadcasted_iota(jnp.int32, (100, 128), 0) * 128
           + lax.broadcasted_iota(jnp.int32, (100, 128), 1))
    lanei = lax.broadcasted_iota(jnp.int32, (1, 128), 1)

    def step(n, carry):
        uc = uwork[...]
        vmax = jnp.max(uc)
        sel = jnp.min(jnp.where(uc == vmax, lin, _KN))
        onehot = lin == sel
        uwork[...] = jnp.where(onehot, -jnp.inf, uc)
        ohf = onehot.astype(jnp.float32)
        vals = [jnp.sum(ohf * r[0]) for r in
                (cx_ref, cy_ref, f00_ref, f10_ref, f01_ref, f11_ref,
                 g00_ref, g10_ref, g01_ref, g11_ref)]
        row = jnp.zeros((1, 128), jnp.float32)
        for c, v in enumerate(vals):
            row = jnp.where(lanei == c, v, row)
        tab[pl.ds(n, 1), :] = row
        return carry

    lax.fori_loop(0, _NB, step, 0)
    t = tab[...]
    rows = lax.broadcasted_iota(jnp.int32, (_NPTS, 1), 0)
    covfull = jnp.concatenate(
        [jnp.zeros((_NPTS - 32, 128), jnp.float32), cov_ref[0]], axis=0)
    t = jnp.where(rows < _NB, t, covfull)
    pts_ref[0] = t
    piota = lax.broadcasted_iota(jnp.int32, (1, _P), 1).astype(jnp.float32)
    s_mat = (t[:, 6:7] * (piota == t[:, 2:3]).astype(jnp.float32)
             + t[:, 7:8] * (piota == t[:, 3:4]).astype(jnp.float32)
             + t[:, 8:9] * (piota == t[:, 4:5]).astype(jnp.float32)
             + t[:, 9:10] * (piota == t[:, 5:6]).astype(jnp.float32))
    fr = jnp.dot(s_mat, mft_ref[0], preferred_element_type=jnp.float32, precision=lax.Precision.HIGHEST)
    h = jnp.maximum(jnp.dot(fr, w1_ref[...], preferred_element_type=jnp.float32, precision=lax.Precision.HIGHEST), 0.0)
    h = jnp.maximum(jnp.dot(h, w2_ref[...], preferred_element_type=jnp.float32, precision=lax.Precision.HIGHEST), 0.0)
    h = jnp.maximum(jnp.dot(h, w3_ref[...], preferred_element_type=jnp.float32, precision=lax.Precision.HIGHEST), 0.0)
    rend_ref[0] = (jnp.dot(h, w4_ref[...], preferred_element_type=jnp.float32, precision=lax.Precision.HIGHEST)
                   + b4_ref[0, 0])


def _tc_select_mlp(u3, cand_f32, covtab, mft, w1t, w2t, w3t, w4t, b4):
    pb = lambda *dims: pl.BlockSpec((1,) + dims, lambda b: (b,) + (0,) * len(dims))
    shared = lambda *dims: pl.BlockSpec(dims, lambda b: (0,) * len(dims))
    in_specs = ([pb(100, 128)] * 11 + [pb(32, 128), pb(_P, 640)]
                + [shared(640, 256), shared(256, 256), shared(256, 256),
                   shared(256, 128), shared(1, 1)])
    out_specs = [pb(_NPTS, 128), pb(_NPTS, 128)]
    return pl.pallas_call(
        _tc_body,
        grid=(_B,),
        in_specs=in_specs,
        out_specs=out_specs,
        out_shape=[jax.ShapeDtypeStruct((_B, _NPTS, 128), jnp.float32),
                   jax.ShapeDtypeStruct((_B, _NPTS, 128), jnp.float32)],
        scratch_shapes=[pltpu.VMEM((100, 128), jnp.float32),
                        pltpu.VMEM((_NPTS, 128), jnp.float32)],
    )(u3, *cand_f32, covtab, mft, w1t, w2t, w3t, w4t, b4)


def kernel(x, feature, mask, W1, W2, W3, W4, b4):
    del x
    sm = jax.nn.softmax(mask, axis=1)
    srt = -jnp.sort(-sm, axis=1)
    s0 = srt[:, 0].reshape(_B, _P)
    s1 = srt[:, 1].reshape(_B, _P)
    i00, i10, i01, i11 = (jnp.asarray(v) for v in _CAND_I32)
    w00, w10, w01, w11 = (jnp.asarray(v) for v in _CAND_W)

    u = _sc_uncertainty(s0, s1, i00, i10, i01, i11, w00, w10, w01, w11)

    maskt = mask.reshape(_B, 3, _P).transpose(0, 2, 1)
    featt = feature.reshape(_B, 512, _P).transpose(0, 2, 1)
    mft = jnp.concatenate(
        [maskt, featt, jnp.zeros((_B, _P, 640 - 515), jnp.float32)], axis=2)
    w1t = jnp.concatenate(
        [W1.T, jnp.zeros((640 - 515, 256), jnp.float32)], axis=0)
    w4t = jnp.concatenate(
        [W4.T, jnp.zeros((256, 127), jnp.float32)], axis=1)
    cand = [jnp.asarray(v) for v in _CAND_F32]

    pts_tab, rend_tab = _tc_select_mlp(
        u.reshape(_B, 100, 128), cand, jnp.asarray(_COVTAB), mft,
        w1t, W2.T, W3.T, w4t, b4.reshape(1, 1))

    points = pts_tab[:, :, 0:2]
    rend = rend_tab[:, :, 0].reshape(_B, 1, _NPTS)
    return rend, points, mask


# speedup vs baseline: 9.9044x; 1.0160x over previous
import functools

import jax
import jax.numpy as jnp
import numpy as np
from jax import lax
from jax.experimental import pallas as pl
from jax.experimental.pallas import tpu as pltpu
from jax.experimental.pallas import tpu_sc as plsc

_B = 8
_KN = 12800
_NB = 230
_NCOV = 26
_NPTS = 256
_HW = 64
_P = _HW * _HW


def _corner_data(pts):
    g = 2.0 * pts - 1.0
    x = ((g[..., 0] + 1.0) * _HW - 1.0) / 2.0
    y = ((g[..., 1] + 1.0) * _HW - 1.0) / 2.0
    x0 = jnp.floor(x)
    x1 = x0 + 1.0
    y0 = jnp.floor(y)
    y1 = y0 + 1.0
    wx1 = x - x0
    wx0 = 1.0 - wx1
    wy1 = y - y0
    wy0 = 1.0 - wy1
    idxs, wts = [], []
    for xi, yi, wx, wy in ((x0, y0, wx0, wy0), (x1, y0, wx1, wy0),
                           (x0, y1, wx0, wy1), (x1, y1, wx1, wy1)):
        valid = ((xi >= 0) & (xi <= _HW - 1) & (yi >= 0) & (yi <= _HW - 1))
        xi_c = jnp.clip(xi, 0, _HW - 1).astype(jnp.int32)
        yi_c = jnp.clip(yi, 0, _HW - 1).astype(jnp.int32)
        idxs.append(yi_c * _HW + xi_c)
        wts.append(jnp.where(valid, wx * wy, 0.0))
    return idxs, wts


def _build_consts():
    key = jax.random.key(42)
    k1, k2 = jax.random.split(key)
    over = jax.random.uniform(k1, (_B, _KN, 2), dtype=jnp.float32)
    coverage = jax.random.uniform(k2, (_B, _NCOV, 2), dtype=jnp.float32)
    ci, cw = _corner_data(over)
    vi, vw = _corner_data(coverage)
    covtab = np.zeros((_B, 32, 128), np.float32)
    cols = [coverage[..., 0], coverage[..., 1]] + \
           [v.astype(jnp.float32) for v in vi] + vw
    for c, v in enumerate(cols):
        covtab[:, 6:32, c] = np.asarray(v)
    cand_f32 = [np.asarray(over[..., 0]).reshape(_B, 100, 128),
                np.asarray(over[..., 1]).reshape(_B, 100, 128)] + \
               [np.asarray(v.astype(jnp.float32)).reshape(_B, 100, 128) for v in ci] + \
               [np.asarray(v).reshape(_B, 100, 128) for v in cw]
    cand_i32 = [np.asarray(v).astype(np.int32) for v in ci]
    cand_w = [np.asarray(v) for v in cw]
    return cand_i32, cand_w, cand_f32, covtab


_CAND_I32, _CAND_W, _CAND_F32, _COVTAB = _build_consts()

_CHUNK = _KN // 4


def _sc_uncertainty(s0, s1, i00, i10, i01, i11, w00, w10, w01, w11):
    mesh = plsc.VectorSubcoreMesh(core_axis_name="c", subcore_axis_name="s")

    @functools.partial(
        pl.kernel, mesh=mesh,
        compiler_params=pltpu.CompilerParams(needs_layout_passes=False),
        out_type=jax.ShapeDtypeStruct((_B, _KN), jnp.float32),
        scratch_types=[
            pltpu.VMEM((_P,), jnp.float32), pltpu.VMEM((_P,), jnp.float32),
            pltpu.VMEM((_CHUNK,), jnp.int32), pltpu.VMEM((_CHUNK,), jnp.int32),
            pltpu.VMEM((_CHUNK,), jnp.int32), pltpu.VMEM((_CHUNK,), jnp.int32),
            pltpu.VMEM((_CHUNK,), jnp.float32), pltpu.VMEM((_CHUNK,), jnp.float32),
            pltpu.VMEM((_CHUNK,), jnp.float32), pltpu.VMEM((_CHUNK,), jnp.float32),
            pltpu.VMEM((_CHUNK,), jnp.float32),
        ])
    def k(s0_h, s1_h, i00_h, i10_h, i01_h, i11_h, w00_h, w10_h, w01_h, w11_h,
          u_h, s0_v, s1_v, i00_v, i10_v, i01_v, i11_v,
          w00_v, w10_v, w01_v, w11_v, u_v):
        wid = lax.axis_index("s") * 2 + lax.axis_index("c")
        b = wid // 4
        off = (wid % 4) * _CHUNK
        pltpu.sync_copy(s0_h.at[b], s0_v)
        pltpu.sync_copy(s1_h.at[b], s1_v)
        for src, dst in ((i00_h, i00_v), (i10_h, i10_v), (i01_h, i01_v),
                         (i11_h, i11_v), (w00_h, w00_v), (w10_h, w10_v),
                         (w01_h, w01_v), (w11_h, w11_v)):
            pltpu.sync_copy(src.at[b, pl.ds(off, _CHUNK)], dst)

        def body(t, carry):
            s = pl.ds(t * 16, 16)
            j00, j10, j01, j11 = i00_v[s], i10_v[s], i01_v[s], i11_v[s]
            a = plsc.load_gather(s0_v, [j00])
            bq = plsc.load_gather(s0_v, [j10])
            c = plsc.load_gather(s0_v, [j01])
            d = plsc.load_gather(s0_v, [j11])
            e = plsc.load_gather(s1_v, [j00])
            f = plsc.load_gather(s1_v, [j10])
            g = plsc.load_gather(s1_v, [j01])
            h = plsc.load_gather(s1_v, [j11])
            q00, q10, q01, q11 = w00_v[s], w10_v[s], w01_v[s], w11_v[s]
            og0 = ((a * q00 + bq * q10) + c * q01) + d * q11
            og1 = ((e * q00 + f * q10) + g * q01) + h * q11
            u_v[s] = og1 - og0
            return carry

        lax.fori_loop(0, _CHUNK // 16, body, 0)
        pltpu.sync_copy(u_v, u_h.at[b, pl.ds(off, _CHUNK)])

    return k(s0, s1, i00, i10, i01, i11, w00, w10, w01, w11)


def _tc_body(u_ref, cx_ref, cy_ref, f00_ref, f10_ref, f01_ref, f11_ref,
             g00_ref, g10_ref, g01_ref, g11_ref, cov_ref, mft_ref,
             w1_ref, w2_ref, w3_ref, w4_ref, b4_ref,
             pts_ref, rend_ref, uwork, tab):
    uwork[...] = u_ref[0]
    lin = (lax.broadcasted_iota(jnp.int32, (100, 128), 0) * 128
           + lax.broadcasted_iota(jnp.int32, (100, 128), 1))
    lanei = lax.broadcasted_iota(jnp.int32, (1, 128), 1)

    def step(n, carry):
        uc = uwork[...]
        vmax = jnp.max(uc)
        sel = jnp.min(jnp.where(uc == vmax, lin, _KN))
        onehot = lin == sel
        uwork[...] = jnp.where(onehot, -jnp.inf, uc)
        ohf = onehot.astype(jnp.float32)
        vals = [jnp.sum(ohf * r[0]) for r in
                (cx_ref, cy_ref, f00_ref, f10_ref, f01_ref, f11_ref,
                 g00_ref, g10_ref, g01_ref, g11_ref)]
        row = jnp.zeros((1, 128), jnp.float32)
        for c, v in enumerate(vals):
            row = jnp.where(lanei == c, v, row)
        tab[pl.ds(n, 1), :] = row
        return carry

    lax.fori_loop(0, _NB, step, 0)
    t = tab[...]
    rows = lax.broadcasted_iota(jnp.int32, (_NPTS, 1), 0)
    covfull = jnp.concatenate(
        [jnp.zeros((_NPTS - 32, 128), jnp.float32), cov_ref[0]], axis=0)
    t = jnp.where(rows < _NB, t, covfull)
    pts_ref[0] = t
    piota = lax.broadcasted_iota(jnp.int32, (1, _P), 1).astype(jnp.float32)
    s_mat = (t[:, 6:7] * (piota == t[:, 2:3]).astype(jnp.float32)
             + t[:, 7:8] * (piota == t[:, 3:4]).astype(jnp.float32)
             + t[:, 8:9] * (piota == t[:, 4:5]).astype(jnp.float32)
             + t[:, 9:10] * (piota == t[:, 5:6]).astype(jnp.float32))
    fr = jnp.dot(s_mat, mft_ref[0], preferred_element_type=jnp.float32, precision=lax.Precision.HIGHEST)
    h = jnp.maximum(jnp.dot(fr, w1_ref[...], preferred_element_type=jnp.float32), 0.0)
    h = jnp.maximum(jnp.dot(h, w2_ref[...], preferred_element_type=jnp.float32), 0.0)
    h = jnp.maximum(jnp.dot(h, w3_ref[...], preferred_element_type=jnp.float32), 0.0)
    rend_ref[0] = (jnp.dot(h, w4_ref[...], preferred_element_type=jnp.float32)
                   + b4_ref[0, 0])


def _tc_select_mlp(u3, cand_f32, covtab, mft, w1t, w2t, w3t, w4t, b4):
    pb = lambda *dims: pl.BlockSpec((1,) + dims, lambda b: (b,) + (0,) * len(dims))
    shared = lambda *dims: pl.BlockSpec(dims, lambda b: (0,) * len(dims))
    in_specs = ([pb(100, 128)] * 11 + [pb(32, 128), pb(_P, 640)]
                + [shared(640, 256), shared(256, 256), shared(256, 256),
                   shared(256, 128), shared(1, 1)])
    out_specs = [pb(_NPTS, 128), pb(_NPTS, 128)]
    return pl.pallas_call(
        _tc_body,
        grid=(_B,),
        in_specs=in_specs,
        out_specs=out_specs,
        out_shape=[jax.ShapeDtypeStruct((_B, _NPTS, 128), jnp.float32),
                   jax.ShapeDtypeStruct((_B, _NPTS, 128), jnp.float32)],
        scratch_shapes=[pltpu.VMEM((100, 128), jnp.float32),
                        pltpu.VMEM((_NPTS, 128), jnp.float32)],
    )(u3, *cand_f32, covtab, mft, w1t, w2t, w3t, w4t, b4)


def kernel(x, feature, mask, W1, W2, W3, W4, b4):
    del x
    sm = jax.nn.softmax(mask, axis=1)
    srt = -jnp.sort(-sm, axis=1)
    s0 = srt[:, 0].reshape(_B, _P)
    s1 = srt[:, 1].reshape(_B, _P)
    i00, i10, i01, i11 = (jnp.asarray(v) for v in _CAND_I32)
    w00, w10, w01, w11 = (jnp.asarray(v) for v in _CAND_W)

    u = _sc_uncertainty(s0, s1, i00, i10, i01, i11, w00, w10, w01, w11)

    maskt = mask.reshape(_B, 3, _P).transpose(0, 2, 1)
    featt = feature.reshape(_B, 512, _P).transpose(0, 2, 1)
    mft = jnp.concatenate(
        [maskt, featt, jnp.zeros((_B, _P, 640 - 515), jnp.float32)], axis=2)
    w1t = jnp.concatenate(
        [W1.T, jnp.zeros((640 - 515, 256), jnp.float32)], axis=0)
    w4t = jnp.concatenate(
        [W4.T, jnp.zeros((256, 127), jnp.float32)], axis=1)
    cand = [jnp.asarray(v) for v in _CAND_F32]

    pts_tab, rend_tab = _tc_select_mlp(
        u.reshape(_B, 100, 128), cand, jnp.asarray(_COVTAB), mft,
        w1t, W2.T, W3.T, w4t, b4.reshape(1, 1))

    points = pts_tab[:, :, 0:2]
    rend = rend_tab[:, :, 0].reshape(_B, 1, _NPTS)
    return rend, points, mask
